# baseline (speedup 1.0000x reference)
import functools

import jax
import jax.numpy as jnp
from jax import lax
from jax.experimental import pallas as pl
from jax.experimental.pallas import tpu as pltpu
from jax.experimental.pallas import tpu_sc as plsc

_N = 10000
_D = 128
_E = 320000
_K = 50
_NE = 65536

_NC, _NS = 2, 16
_NW = _NC * _NS
_EPW = _E // _NW
_CHUNK = 128
_ECH = 80
_NCHUNK = 126
_EPAD = _NCHUNK * _ECH
_NAGG = 10016
_NDEG = 10112
_R = 2000
_NBLK = _N // _R
_NEPW = _NE // _NW

@functools.cache
def _mesh():
    return plsc.VectorSubcoreMesh(
        core_axis_name="c", subcore_axis_name="s",
        num_cores=_NC, num_subcores=_NS)


def _sc_deg_body(dst_hbm, out_hbm, idx_v, deg_v):
    c = lax.axis_index("c")
    s = lax.axis_index("s")
    wid = c * _NS + s
    pltpu.sync_copy(dst_hbm.at[wid], idx_v)

    def zero(i, carry):
        deg_v[pl.ds(i * 16, 16)] = jnp.zeros((16,), jnp.float32)
        return carry

    lax.fori_loop(0, _NDEG // 16, zero, 0)
    ones = jnp.ones((16,), jnp.float32)

    def acc(i, carry):
        plsc.addupdate_scatter(deg_v, [idx_v[i, :]], ones)
        return carry

    lax.fori_loop(0, _EPAD // 16, acc, 0)
    pltpu.sync_copy(deg_v, out_hbm.at[wid])


@functools.cache
def _sc_deg():
    return pl.kernel(
        _sc_deg_body,
        out_type=jax.ShapeDtypeStruct((_NW, _NDEG), jnp.float32),
        mesh=_mesh(),
        compiler_params=pltpu.CompilerParams(use_tc_tiling_on_sc=False, needs_layout_passes=False),
        scratch_types=[
            pltpu.VMEM((_EPAD // 16, 16), jnp.int32),
            pltpu.VMEM((_NDEG,), jnp.float32),
        ],
    )


def _sc_agg_body(hn_hbm, src_hbm, dst_hbm, zeros_hbm, out_hbm,
                 srci_v, dsti_v, r0, r1, r2, agg_sh, s0, s1, s2):
    c = lax.axis_index("c")
    s = lax.axis_index("s")
    wid = c * _NS + s
    rpt = _NAGG // _NS
    pltpu.sync_copy(zeros_hbm.at[pl.ds(s * rpt, rpt)],
                    agg_sh.at[pl.ds(s * rpt, rpt)])
    pltpu.sync_copy(src_hbm.at[wid], srci_v)
    pltpu.sync_copy(dst_hbm.at[wid], dsti_v)
    plsc.subcore_barrier()

    bufs = (r0, r1, r2)
    sems = (s0, s1, s2)
    nbuf = 3
    for b in range(nbuf):
        pltpu.async_copy(hn_hbm.at[srci_v.at[b]], bufs[b], sems[b])

    def step(i, carry):
        for b in range(nbuf):
            j = nbuf * i + b
            pltpu.make_async_copy(
                hn_hbm.at[srci_v.at[j]], bufs[b], sems[b]).wait()
            pltpu.sync_copy(bufs[b], agg_sh.at[dsti_v.at[j]], add=True)

            @pl.when(j + nbuf < _NCHUNK)
            def _():
                pltpu.async_copy(
                    hn_hbm.at[srci_v.at[j + nbuf]], bufs[b], sems[b])
        return carry

    lax.fori_loop(0, _NCHUNK // nbuf, step, 0)
    plsc.subcore_barrier()
    opt = _N // _NS
    pltpu.sync_copy(agg_sh.at[pl.ds(s * opt, opt)],
                    out_hbm.at[c, pl.ds(s * opt, opt)])


@functools.cache
def _sc_agg():
    return pl.kernel(
        _sc_agg_body,
        out_type=jax.ShapeDtypeStruct((_NC, _N, _D), jnp.float32),
        mesh=_mesh(),
        compiler_params=pltpu.CompilerParams(use_tc_tiling_on_sc=False, needs_layout_passes=False),
        scratch_types=[
            pltpu.VMEM((_NCHUNK, _ECH), jnp.int32),
            pltpu.VMEM((_NCHUNK, _ECH), jnp.int32),
            pltpu.VMEM((_ECH, _D), jnp.float32),
            pltpu.VMEM((_ECH, _D), jnp.float32),
            pltpu.VMEM((_ECH, _D), jnp.float32),
            pltpu.VMEM_SHARED((_NAGG, _D), jnp.float32),
            pltpu.SemaphoreType.DMA,
            pltpu.SemaphoreType.DMA,
            pltpu.SemaphoreType.DMA,
        ],
    )


def _sc_pairs_body(x_hbm, e0_hbm, e1_hbm, g0_hbm, g1_hbm,
                   i0_v, i1_v, buf0, buf1, sem0, sem1):
    c = lax.axis_index("c")
    s = lax.axis_index("s")
    wid = c * _NS + s
    pltpu.sync_copy(e0_hbm.at[wid], i0_v)
    pltpu.sync_copy(e1_hbm.at[wid], i1_v)
    base = wid * _NEPW
    nchunk = _NEPW // _CHUNK
    pltpu.async_copy(x_hbm.at[i0_v.at[0]], buf0, sem0)

    def step(j, carry):
        pltpu.async_copy(x_hbm.at[i1_v.at[j]], buf1, sem1)
        pltpu.make_async_copy(x_hbm.at[i0_v.at[j]], buf0, sem0).wait()
        pltpu.sync_copy(buf0, g0_hbm.at[pl.ds(base + j * _CHUNK, _CHUNK)])

        @pl.when(j < nchunk - 1)
        def _():
            pltpu.async_copy(x_hbm.at[i0_v.at[j + 1]], buf0, sem0)

        pltpu.make_async_copy(x_hbm.at[i1_v.at[j]], buf1, sem1).wait()
        pltpu.sync_copy(buf1, g1_hbm.at[pl.ds(base + j * _CHUNK, _CHUNK)])
        return carry

    lax.fori_loop(0, nchunk, step, 0)


@functools.cache
def _sc_pairs():
    return pl.kernel(
        _sc_pairs_body,
        out_type=(
            jax.ShapeDtypeStruct((_NE, _D), jnp.float32),
            jax.ShapeDtypeStruct((_NE, _D), jnp.float32),
        ),
        mesh=_mesh(),
        compiler_params=pltpu.CompilerParams(use_tc_tiling_on_sc=False, needs_layout_passes=False),
        scratch_types=[
            pltpu.VMEM((_NEPW // _CHUNK, _CHUNK), jnp.int32),
            pltpu.VMEM((_NEPW // _CHUNK, _CHUNK), jnp.int32),
            pltpu.VMEM((_CHUNK, _D), jnp.float32),
            pltpu.VMEM((_CHUNK, _D), jnp.float32),
            pltpu.SemaphoreType.DMA,
            pltpu.SemaphoreType.DMA,
        ],
    )


def _bn_apply(y, sums, g, b):
    m = sums[0:1, :] / _N
    var = sums[1:2, :] / _N - m * m
    return (y - m) * lax.rsqrt(var + 1e-5) * g + b


def _attn_core(x, Wa, ba, ut_ref, s_ref, cs_ref, step):
    tmp = jax.nn.relu(jnp.dot(x, Wa, preferred_element_type=jnp.float32) + ba)
    U = tmp[:, 0:_K]
    V = tmp[:, _K:2 * _K]
    Z = tmp[:, 2 * _K:3 * _K]
    T = tmp[:, 3 * _K:4 * _K]
    ut_ref[...] = jnp.concatenate([U, T], axis=1)
    s_c = lax.dot_general(V, Z, (((0,), (0,)), ((), ())),
                          preferred_element_type=jnp.float32)
    cs_c = jnp.concatenate(
        [jnp.sum(U, axis=0)[None, :], jnp.sum(V, axis=0)[None, :]], axis=0)

    @pl.when(step == 0)
    def _():
        s_ref[...] = jnp.zeros_like(s_ref)
        cs_ref[...] = jnp.zeros_like(cs_ref)

    s_ref[...] += s_c
    cs_ref[...] += cs_c


def _conv0_body(x_ref, degpT_ref, Wc_ref, hn_ref, dinv_ref):
    d = jnp.sum(degpT_ref[...], axis=1) + 1.0
    dinv_c = lax.rsqrt(d)[:, None]
    dinv_ref[...] = dinv_c
    h = jnp.dot(x_ref[...], Wc_ref[...], preferred_element_type=jnp.float32)
    hn_ref[...] = h * dinv_c


def _attn0_body(x_ref, Wa_ref, ba_ref, ut_ref, s_ref, cs_ref):
    _attn_core(x_ref[...], Wa_ref[...], ba_ref[...],
               ut_ref, s_ref, cs_ref, pl.program_id(0))


def _conv1_body(y_ref, bnstats_ref, bng_ref, bnb_ref, dinv_ref, Wc_ref,
                hn_ref):
    x = _bn_apply(y_ref[...], bnstats_ref[...], bng_ref[...], bnb_ref[...])
    h = jnp.dot(x, Wc_ref[...], preferred_element_type=jnp.float32)
    hn_ref[...] = h * dinv_ref[...]


def _attn1_body(y_ref, bnstats_ref, bng_ref, bnb_ref, Wa_ref, ba_ref,
                ut_ref, s_ref, cs_ref):
    x = _bn_apply(y_ref[...], bnstats_ref[...], bng_ref[...], bnb_ref[...])
    _attn_core(x, Wa_ref[...], ba_ref[...],
               ut_ref, s_ref, cs_ref, pl.program_id(0))


_attn_out_specs = [
    pl.BlockSpec((_R, 2 * _K), lambda i: (i, 0)),
    pl.BlockSpec((_K, _K), lambda i: (0, 0)),
    pl.BlockSpec((2, _K), lambda i: (0, 0)),
]
_attn_out_shape = [
    jax.ShapeDtypeStruct((_N, 2 * _K), jnp.float32),
    jax.ShapeDtypeStruct((_K, _K), jnp.float32),
    jax.ShapeDtypeStruct((2, _K), jnp.float32),
]


def _tc_conv0(x, degpT, Wc):
    return pl.pallas_call(
        _conv0_body,
        grid=(_NBLK,),
        in_specs=[
            pl.BlockSpec((_R, _D), lambda i: (i, 0)),
            pl.BlockSpec((_R, _NW), lambda i: (i, 0)),
            pl.BlockSpec((_D, _D), lambda i: (0, 0)),
        ],
        out_specs=[
            pl.BlockSpec((_R, _D), lambda i: (i, 0)),
            pl.BlockSpec((_R, 1), lambda i: (i, 0)),
        ],
        out_shape=[
            jax.ShapeDtypeStruct((_N, _D), jnp.float32),
            jax.ShapeDtypeStruct((_N, 1), jnp.float32),
        ],
    )(x, degpT, Wc)


def _tc_attn0(x, Wa, ba):
    return pl.pallas_call(
        _attn0_body,
        grid=(_NBLK,),
        in_specs=[
            pl.BlockSpec((_R, _D), lambda i: (i, 0)),
            pl.BlockSpec((_D, 4 * _K), lambda i: (0, 0)),
            pl.BlockSpec((1, 4 * _K), lambda i: (0, 0)),
        ],
        out_specs=_attn_out_specs,
        out_shape=_attn_out_shape,
    )(x, Wa, ba)


def _tc_conv1(y, bnstats, bng, bnb, dinv, Wc):
    return pl.pallas_call(
        _conv1_body,
        grid=(_NBLK,),
        in_specs=[
            pl.BlockSpec((_R, _D), lambda i: (i, 0)),
            pl.BlockSpec((2, _D), lambda i: (0, 0)),
            pl.BlockSpec((1, _D), lambda i: (0, 0)),
            pl.BlockSpec((1, _D), lambda i: (0, 0)),
            pl.BlockSpec((_R, 1), lambda i: (i, 0)),
            pl.BlockSpec((_D, _D), lambda i: (0, 0)),
        ],
        out_specs=pl.BlockSpec((_R, _D), lambda i: (i, 0)),
        out_shape=jax.ShapeDtypeStruct((_N, _D), jnp.float32),
    )(y, bnstats, bng, bnb, dinv, Wc)


def _tc_attn1(y, bnstats, bng, bnb, Wa, ba):
    return pl.pallas_call(
        _attn1_body,
        grid=(_NBLK,),
        in_specs=[
            pl.BlockSpec((_R, _D), lambda i: (i, 0)),
            pl.BlockSpec((2, _D), lambda i: (0, 0)),
            pl.BlockSpec((1, _D), lambda i: (0, 0)),
            pl.BlockSpec((1, _D), lambda i: (0, 0)),
            pl.BlockSpec((_D, 4 * _K), lambda i: (0, 0)),
            pl.BlockSpec((1, 4 * _K), lambda i: (0, 0)),
        ],
        out_specs=_attn_out_specs,
        out_shape=_attn_out_shape,
    )(y, bnstats, bng, bnb, Wa, ba)


def _combine_core(ut, s, cs, aggp, hn, dinv_c, bc, Wd, bd):
    norm = jnp.sum(cs[0, :] * cs[1, :]) / _N + 1e-6
    xl = jax.nn.relu(dinv_c * (aggp[0] + aggp[1] + hn) + bc)
    U = ut[:, 0:_K]
    T = ut[:, _K:2 * _K]
    sw = jnp.dot(s, Wd[0:_K, :], preferred_element_type=jnp.float32)
    y = (jnp.dot(U, sw, preferred_element_type=jnp.float32) / norm
         + jnp.dot(T, Wd[_K:2 * _K, :], preferred_element_type=jnp.float32)
         + jnp.dot(xl, Wd[2 * _K:2 * _K + _D, :],
                   preferred_element_type=jnp.float32)
         + bd)
    return y


def _comb0_body(ut_ref, s_ref, cs_ref, aggp_ref, hn_ref, dinv_ref,
                bc_ref, Wd_ref, bd_ref, y_ref, bn_ref):
    y = jax.nn.relu(_combine_core(
        ut_ref[...], s_ref[...], cs_ref[...], aggp_ref[...], hn_ref[...],
        dinv_ref[...], bc_ref[...], Wd_ref[...], bd_ref[...]))
    y_ref[...] = y
    stats = jnp.concatenate(
        [jnp.sum(y, axis=0)[None, :], jnp.sum(y * y, axis=0)[None, :]], axis=0)

    @pl.when(pl.program_id(0) == 0)
    def _():
        bn_ref[...] = jnp.zeros_like(bn_ref)

    bn_ref[...] += stats


def _comb1_body(ut_ref, s_ref, cs_ref, aggp_ref, hn_ref, dinv_ref,
                bc_ref, Wd_ref, bd_ref, y_ref):
    y_ref[...] = _combine_core(
        ut_ref[...], s_ref[...], cs_ref[...], aggp_ref[...], hn_ref[...],
        dinv_ref[...], bc_ref[...], Wd_ref[...], bd_ref[...])


_comb_in_specs = [
    pl.BlockSpec((_R, 2 * _K), lambda i: (i, 0)),
    pl.BlockSpec((_K, _K), lambda i: (0, 0)),
    pl.BlockSpec((2, _K), lambda i: (0, 0)),
    pl.BlockSpec((_NC, _R, _D), lambda i: (0, i, 0)),
    pl.BlockSpec((_R, _D), lambda i: (i, 0)),
    pl.BlockSpec((_R, 1), lambda i: (i, 0)),
    pl.BlockSpec((1, _D), lambda i: (0, 0)),
    pl.BlockSpec((2 * _K + _D, _D), lambda i: (0, 0)),
    pl.BlockSpec((1, _D), lambda i: (0, 0)),
]


def _tc_comb0(ut, s, cs, aggp, hn, dinv, bc, Wd, bd):
    return pl.pallas_call(
        _comb0_body,
        grid=(_NBLK,),
        in_specs=_comb_in_specs,
        out_specs=[
            pl.BlockSpec((_R, _D), lambda i: (i, 0)),
            pl.BlockSpec((2, _D), lambda i: (0, 0)),
        ],
        out_shape=[
            jax.ShapeDtypeStruct((_N, _D), jnp.float32),
            jax.ShapeDtypeStruct((2, _D), jnp.float32),
        ],
    )(ut, s, cs, aggp, hn, dinv, bc, Wd, bd)


def _tc_comb1(ut, s, cs, aggp, hn, dinv, bc, Wd, bd):
    return pl.pallas_call(
        _comb1_body,
        grid=(_NBLK,),
        in_specs=_comb_in_specs,
        out_specs=pl.BlockSpec((_R, _D), lambda i: (i, 0)),
        out_shape=jax.ShapeDtypeStruct((_N, _D), jnp.float32),
    )(ut, s, cs, aggp, hn, dinv, bc, Wd, bd)


def _pred_body(g0_ref, g1_ref, W0_ref, b0_ref, W1_ref, b1_ref, out_ref):
    m = g0_ref[...] * g1_ref[...]
    h = jax.nn.relu(jnp.dot(m, W0_ref[...], preferred_element_type=jnp.float32)
                    + b0_ref[...])
    o = jnp.dot(h, W1_ref[...], preferred_element_type=jnp.float32) + b1_ref[...]
    out_ref[...] = jax.nn.sigmoid(o)


_BP = 4096


def _tc_pred(g0, g1, W0, b0, W1, b1):
    return pl.pallas_call(
        _pred_body,
        grid=(_NE // _BP,),
        in_specs=[
            pl.BlockSpec((_BP, _D), lambda i: (i, 0)),
            pl.BlockSpec((_BP, _D), lambda i: (i, 0)),
            pl.BlockSpec((_D, _D), lambda i: (0, 0)),
            pl.BlockSpec((1, _D), lambda i: (0, 0)),
            pl.BlockSpec((_D, 1), lambda i: (0, 0)),
            pl.BlockSpec((1, 1), lambda i: (0, 0)),
        ],
        out_specs=pl.BlockSpec((_BP, 1), lambda i: (i, 0)),
        out_shape=jax.ShapeDtypeStruct((_NE, 1), jnp.float32),
    )(g0, g1, W0, b0, W1, b1)


def kernel(adj_t, edges, emb, conv_W0, conv_b0, attn_W0, attn_b0, dr_W0,
           dr_b0, bn_g0, bn_b0, conv_W1, conv_b1, attn_W1, attn_b1, dr_W1,
           dr_b1, pred_W0, pred_b0, pred_W1, pred_b1):
    pad = _EPAD - _EPW
    srcp = jnp.concatenate(
        [adj_t[0].reshape(_NW, _EPW),
         jnp.zeros((_NW, pad), jnp.int32)], axis=1)
    dstp = jnp.concatenate(
        [adj_t[1].reshape(_NW, _EPW),
         jnp.full((_NW, pad), _N, jnp.int32)], axis=1)
    srcI = srcp.reshape(_NW, _NCHUNK, _ECH)
    dstI = dstp.reshape(_NW, _NCHUNK, _ECH)
    dst16 = dstp.reshape(_NW, _EPAD // 16, 16)
    e0I = edges[0].reshape(_NW, _NEPW // _CHUNK, _CHUNK)
    e1I = edges[1].reshape(_NW, _NEPW // _CHUNK, _CHUNK)
    zerosA = jnp.zeros((_NAGG, _D), jnp.float32)

    bc0 = conv_b0.reshape(1, _D)
    bc1 = conv_b1.reshape(1, _D)
    ba0 = attn_b0.reshape(1, 4 * _K)
    ba1 = attn_b1.reshape(1, 4 * _K)
    bd0 = dr_b0.reshape(1, _D)
    bd1 = dr_b1.reshape(1, _D)
    bng = bn_g0.reshape(1, _D)
    bnb = bn_b0.reshape(1, _D)
    bp0 = pred_b0.reshape(1, _D)
    bp1 = pred_b1.reshape(1, 1)

    degp = _sc_deg()(dst16)

    hn0, dinv = _tc_conv0(emb, degp.T, conv_W0)
    aggp0 = _sc_agg()(hn0, srcI, dstI, zerosA)
    ut0, s0, cs0 = _tc_attn0(emb, attn_W0, ba0)
    y0, bnstats = _tc_comb0(ut0, s0, cs0, aggp0, hn0, dinv, bc0, dr_W0, bd0)

    hn1 = _tc_conv1(y0, bnstats, bng, bnb, dinv, conv_W1)
    aggp1 = _sc_agg()(hn1, srcI, dstI, zerosA)
    ut1, s1, cs1 = _tc_attn1(y0, bnstats, bng, bnb, attn_W1, ba1)
    xf = _tc_comb1(ut1, s1, cs1, aggp1, hn1, dinv, bc1, dr_W1, bd1)

    g0, g1 = _sc_pairs()(xf, e0I, e1I)
    return _tc_pred(g0, g1, pred_W0, bp0, pred_W1, bp1)


# speedup vs baseline: 1.0084x; 1.0084x over previous
import functools

import jax
import jax.numpy as jnp
from jax import lax
from jax.experimental import pallas as pl
from jax.experimental.pallas import tpu as pltpu
from jax.experimental.pallas import tpu_sc as plsc

_N = 10000
_D = 128
_E = 320000
_K = 50
_NE = 65536

_NC, _NS = 2, 16
_NW = _NC * _NS
_EPW = _E // _NW
_CHUNK = 128
_ECH = 56
_NCHUNK = 180
_EPAD = _NCHUNK * _ECH
_NAGG = 10016
_NDEG = 10112
_R = 2000
_NBLK = _N // _R
_NEPW = _NE // _NW

@functools.cache
def _mesh():
    return plsc.VectorSubcoreMesh(
        core_axis_name="c", subcore_axis_name="s",
        num_cores=_NC, num_subcores=_NS)


def _sc_deg_body(dst_hbm, out_hbm, idx_v, deg_v):
    c = lax.axis_index("c")
    s = lax.axis_index("s")
    wid = c * _NS + s
    pltpu.sync_copy(dst_hbm.at[wid], idx_v)

    def zero(i, carry):
        deg_v[pl.ds(i * 16, 16)] = jnp.zeros((16,), jnp.float32)
        return carry

    lax.fori_loop(0, _NDEG // 16, zero, 0)
    ones = jnp.ones((16,), jnp.float32)

    def acc(i, carry):
        plsc.addupdate_scatter(deg_v, [idx_v[i, :]], ones)
        return carry

    lax.fori_loop(0, _EPAD // 16, acc, 0)
    pltpu.sync_copy(deg_v, out_hbm.at[wid])


@functools.cache
def _sc_deg():
    return pl.kernel(
        _sc_deg_body,
        out_type=jax.ShapeDtypeStruct((_NW, _NDEG), jnp.float32),
        mesh=_mesh(),
        compiler_params=pltpu.CompilerParams(use_tc_tiling_on_sc=False, needs_layout_passes=False),
        scratch_types=[
            pltpu.VMEM((_EPAD // 16, 16), jnp.int32),
            pltpu.VMEM((_NDEG,), jnp.float32),
        ],
    )


def _sc_agg_body(hn_hbm, src_hbm, dst_hbm, zeros_hbm, out_hbm,
                 srci_v, dsti_v, r0, r1, r2, r3, agg_sh, s0, s1, s2, s3):
    c = lax.axis_index("c")
    s = lax.axis_index("s")
    wid = c * _NS + s
    rpt = _NAGG // _NS

    @pl.when(jnp.logical_and(c == 0, s < _NS - 1))
    def _():
        pltpu.sync_copy(hn_hbm.at[pl.ds(s * rpt, rpt)],
                        agg_sh.at[pl.ds(s * rpt, rpt)])

    @pl.when(jnp.logical_and(c == 0, s == _NS - 1))
    def _():
        last = _N - (_NS - 1) * rpt
        pltpu.sync_copy(hn_hbm.at[pl.ds((_NS - 1) * rpt, last)],
                        agg_sh.at[pl.ds((_NS - 1) * rpt, last)])
        pltpu.sync_copy(zeros_hbm.at[pl.ds(0, _NAGG - _N)],
                        agg_sh.at[pl.ds(_N, _NAGG - _N)])

    @pl.when(c == 1)
    def _():
        pltpu.sync_copy(zeros_hbm, agg_sh.at[pl.ds(s * rpt, rpt)])
    pltpu.sync_copy(src_hbm.at[wid], srci_v)
    pltpu.sync_copy(dst_hbm.at[wid], dsti_v)
    plsc.subcore_barrier()

    bufs = (r0, r1, r2, r3)
    sems = (s0, s1, s2, s3)
    nbuf = 4
    for b in range(nbuf):
        pltpu.async_copy(hn_hbm.at[srci_v.at[b]], bufs[b], sems[b])

    def step(i, carry):
        for b in range(nbuf):
            j = nbuf * i + b
            pltpu.make_async_copy(
                hn_hbm.at[srci_v.at[j]], bufs[b], sems[b]).wait()
            pltpu.sync_copy(bufs[b], agg_sh.at[dsti_v.at[j]], add=True)

            @pl.when(j + nbuf < _NCHUNK)
            def _():
                pltpu.async_copy(
                    hn_hbm.at[srci_v.at[j + nbuf]], bufs[b], sems[b])
        return carry

    lax.fori_loop(0, _NCHUNK // nbuf, step, 0)
    plsc.subcore_barrier()
    opt = _N // _NS
    pltpu.sync_copy(agg_sh.at[pl.ds(s * opt, opt)],
                    out_hbm.at[c, pl.ds(s * opt, opt)])


@functools.cache
def _sc_agg():
    return pl.kernel(
        _sc_agg_body,
        out_type=jax.ShapeDtypeStruct((_NC, _N, _D), jnp.float32),
        mesh=_mesh(),
        compiler_params=pltpu.CompilerParams(use_tc_tiling_on_sc=False, needs_layout_passes=False),
        scratch_types=[
            pltpu.VMEM((_NCHUNK, _ECH), jnp.int32),
            pltpu.VMEM((_NCHUNK, _ECH), jnp.int32),
            pltpu.VMEM((_ECH, _D), jnp.float32),
            pltpu.VMEM((_ECH, _D), jnp.float32),
            pltpu.VMEM((_ECH, _D), jnp.float32),
            pltpu.VMEM((_ECH, _D), jnp.float32),
            pltpu.VMEM_SHARED((_NAGG, _D), jnp.float32),
            pltpu.SemaphoreType.DMA,
            pltpu.SemaphoreType.DMA,
            pltpu.SemaphoreType.DMA,
            pltpu.SemaphoreType.DMA,
        ],
    )


def _sc_pairs_body(x_hbm, e0_hbm, e1_hbm, g0_hbm, g1_hbm,
                   i0_v, i1_v, buf0, buf1, sem0, sem1):
    c = lax.axis_index("c")
    s = lax.axis_index("s")
    wid = c * _NS + s
    pltpu.sync_copy(e0_hbm.at[wid], i0_v)
    pltpu.sync_copy(e1_hbm.at[wid], i1_v)
    base = wid * _NEPW
    nchunk = _NEPW // _CHUNK
    pltpu.async_copy(x_hbm.at[i0_v.at[0]], buf0, sem0)

    def step(j, carry):
        pltpu.async_copy(x_hbm.at[i1_v.at[j]], buf1, sem1)
        pltpu.make_async_copy(x_hbm.at[i0_v.at[j]], buf0, sem0).wait()
        pltpu.sync_copy(buf0, g0_hbm.at[pl.ds(base + j * _CHUNK, _CHUNK)])

        @pl.when(j < nchunk - 1)
        def _():
            pltpu.async_copy(x_hbm.at[i0_v.at[j + 1]], buf0, sem0)

        pltpu.make_async_copy(x_hbm.at[i1_v.at[j]], buf1, sem1).wait()
        pltpu.sync_copy(buf1, g1_hbm.at[pl.ds(base + j * _CHUNK, _CHUNK)])
        return carry

    lax.fori_loop(0, nchunk, step, 0)


@functools.cache
def _sc_pairs():
    return pl.kernel(
        _sc_pairs_body,
        out_type=(
            jax.ShapeDtypeStruct((_NE, _D), jnp.float32),
            jax.ShapeDtypeStruct((_NE, _D), jnp.float32),
        ),
        mesh=_mesh(),
        compiler_params=pltpu.CompilerParams(use_tc_tiling_on_sc=False, needs_layout_passes=False),
        scratch_types=[
            pltpu.VMEM((_NEPW // _CHUNK, _CHUNK), jnp.int32),
            pltpu.VMEM((_NEPW // _CHUNK, _CHUNK), jnp.int32),
            pltpu.VMEM((_CHUNK, _D), jnp.float32),
            pltpu.VMEM((_CHUNK, _D), jnp.float32),
            pltpu.SemaphoreType.DMA,
            pltpu.SemaphoreType.DMA,
        ],
    )


def _bn_apply(y, sums, g, b):
    m = sums[0:1, :] / _N
    var = sums[1:2, :] / _N - m * m
    return (y - m) * lax.rsqrt(var + 1e-5) * g + b


def _attn_core(x, Wa, ba, ut_ref, s_ref, cs_ref, step):
    tmp = jax.nn.relu(jnp.dot(x, Wa, preferred_element_type=jnp.float32) + ba)
    U = tmp[:, 0:_K]
    V = tmp[:, _K:2 * _K]
    Z = tmp[:, 2 * _K:3 * _K]
    T = tmp[:, 3 * _K:4 * _K]
    ut_ref[...] = jnp.concatenate([U, T], axis=1)
    s_c = lax.dot_general(V, Z, (((0,), (0,)), ((), ())),
                          preferred_element_type=jnp.float32)
    cs_c = jnp.concatenate(
        [jnp.sum(U, axis=0)[None, :], jnp.sum(V, axis=0)[None, :]], axis=0)

    @pl.when(step == 0)
    def _():
        s_ref[...] = jnp.zeros_like(s_ref)
        cs_ref[...] = jnp.zeros_like(cs_ref)

    s_ref[...] += s_c
    cs_ref[...] += cs_c


def _conv0_body(x_ref, degpT_ref, Wc_ref, hn_ref, dinv_ref):
    d = jnp.sum(degpT_ref[...], axis=1) + 1.0
    dinv_c = lax.rsqrt(d)[:, None]
    dinv_ref[...] = dinv_c
    h = jnp.dot(x_ref[...], Wc_ref[...], preferred_element_type=jnp.float32)
    hn_ref[...] = h * dinv_c


def _attn0_body(x_ref, Wa_ref, ba_ref, ut_ref, s_ref, cs_ref):
    _attn_core(x_ref[...], Wa_ref[...], ba_ref[...],
               ut_ref, s_ref, cs_ref, pl.program_id(0))


def _conv1_body(y_ref, bnstats_ref, bng_ref, bnb_ref, dinv_ref, Wc_ref,
                hn_ref):
    x = _bn_apply(y_ref[...], bnstats_ref[...], bng_ref[...], bnb_ref[...])
    h = jnp.dot(x, Wc_ref[...], preferred_element_type=jnp.float32)
    hn_ref[...] = h * dinv_ref[...]


def _attn1_body(y_ref, bnstats_ref, bng_ref, bnb_ref, Wa_ref, ba_ref,
                ut_ref, s_ref, cs_ref):
    x = _bn_apply(y_ref[...], bnstats_ref[...], bng_ref[...], bnb_ref[...])
    _attn_core(x, Wa_ref[...], ba_ref[...],
               ut_ref, s_ref, cs_ref, pl.program_id(0))


_attn_out_specs = [
    pl.BlockSpec((_R, 2 * _K), lambda i: (i, 0)),
    pl.BlockSpec((_K, _K), lambda i: (0, 0)),
    pl.BlockSpec((2, _K), lambda i: (0, 0)),
]
_attn_out_shape = [
    jax.ShapeDtypeStruct((_N, 2 * _K), jnp.float32),
    jax.ShapeDtypeStruct((_K, _K), jnp.float32),
    jax.ShapeDtypeStruct((2, _K), jnp.float32),
]


def _tc_conv0(x, degpT, Wc):
    return pl.pallas_call(
        _conv0_body,
        grid=(_NBLK,),
        in_specs=[
            pl.BlockSpec((_R, _D), lambda i: (i, 0)),
            pl.BlockSpec((_R, _NW), lambda i: (i, 0)),
            pl.BlockSpec((_D, _D), lambda i: (0, 0)),
        ],
        out_specs=[
            pl.BlockSpec((_R, _D), lambda i: (i, 0)),
            pl.BlockSpec((_R, 1), lambda i: (i, 0)),
        ],
        out_shape=[
            jax.ShapeDtypeStruct((_N, _D), jnp.float32),
            jax.ShapeDtypeStruct((_N, 1), jnp.float32),
        ],
    )(x, degpT, Wc)


def _tc_attn0(x, Wa, ba):
    return pl.pallas_call(
        _attn0_body,
        grid=(_NBLK,),
        in_specs=[
            pl.BlockSpec((_R, _D), lambda i: (i, 0)),
            pl.BlockSpec((_D, 4 * _K), lambda i: (0, 0)),
            pl.BlockSpec((1, 4 * _K), lambda i: (0, 0)),
        ],
        out_specs=_attn_out_specs,
        out_shape=_attn_out_shape,
    )(x, Wa, ba)


def _tc_conv1(y, bnstats, bng, bnb, dinv, Wc):
    return pl.pallas_call(
        _conv1_body,
        grid=(_NBLK,),
        in_specs=[
            pl.BlockSpec((_R, _D), lambda i: (i, 0)),
            pl.BlockSpec((2, _D), lambda i: (0, 0)),
            pl.BlockSpec((1, _D), lambda i: (0, 0)),
            pl.BlockSpec((1, _D), lambda i: (0, 0)),
            pl.BlockSpec((_R, 1), lambda i: (i, 0)),
            pl.BlockSpec((_D, _D), lambda i: (0, 0)),
        ],
        out_specs=pl.BlockSpec((_R, _D), lambda i: (i, 0)),
        out_shape=jax.ShapeDtypeStruct((_N, _D), jnp.float32),
    )(y, bnstats, bng, bnb, dinv, Wc)


def _tc_attn1(y, bnstats, bng, bnb, Wa, ba):
    return pl.pallas_call(
        _attn1_body,
        grid=(_NBLK,),
        in_specs=[
            pl.BlockSpec((_R, _D), lambda i: (i, 0)),
            pl.BlockSpec((2, _D), lambda i: (0, 0)),
            pl.BlockSpec((1, _D), lambda i: (0, 0)),
            pl.BlockSpec((1, _D), lambda i: (0, 0)),
            pl.BlockSpec((_D, 4 * _K), lambda i: (0, 0)),
            pl.BlockSpec((1, 4 * _K), lambda i: (0, 0)),
        ],
        out_specs=_attn_out_specs,
        out_shape=_attn_out_shape,
    )(y, bnstats, bng, bnb, Wa, ba)


def _combine_core(ut, s, cs, aggp, dinv_c, bc, Wd, bd):
    norm = jnp.sum(cs[0, :] * cs[1, :]) / _N + 1e-6
    xl = jax.nn.relu(dinv_c * (aggp[0] + aggp[1]) + bc)
    U = ut[:, 0:_K]
    T = ut[:, _K:2 * _K]
    sw = jnp.dot(s, Wd[0:_K, :], preferred_element_type=jnp.float32)
    y = (jnp.dot(U, sw, preferred_element_type=jnp.float32) / norm
         + jnp.dot(T, Wd[_K:2 * _K, :], preferred_element_type=jnp.float32)
         + jnp.dot(xl, Wd[2 * _K:2 * _K + _D, :],
                   preferred_element_type=jnp.float32)
         + bd)
    return y


def _comb0_body(ut_ref, s_ref, cs_ref, aggp_ref, dinv_ref,
                bc_ref, Wd_ref, bd_ref, y_ref, bn_ref):
    y = jax.nn.relu(_combine_core(
        ut_ref[...], s_ref[...], cs_ref[...], aggp_ref[...],
        dinv_ref[...], bc_ref[...], Wd_ref[...], bd_ref[...]))
    y_ref[...] = y
    stats = jnp.concatenate(
        [jnp.sum(y, axis=0)[None, :], jnp.sum(y * y, axis=0)[None, :]], axis=0)

    @pl.when(pl.program_id(0) == 0)
    def _():
        bn_ref[...] = jnp.zeros_like(bn_ref)

    bn_ref[...] += stats


def _comb1_body(ut_ref, s_ref, cs_ref, aggp_ref, dinv_ref,
                bc_ref, Wd_ref, bd_ref, y_ref):
    y_ref[...] = _combine_core(
        ut_ref[...], s_ref[...], cs_ref[...], aggp_ref[...],
        dinv_ref[...], bc_ref[...], Wd_ref[...], bd_ref[...])


_comb_in_specs = [
    pl.BlockSpec((_R, 2 * _K), lambda i: (i, 0)),
    pl.BlockSpec((_K, _K), lambda i: (0, 0)),
    pl.BlockSpec((2, _K), lambda i: (0, 0)),
    pl.BlockSpec((_NC, _R, _D), lambda i: (0, i, 0)),
    pl.BlockSpec((_R, 1), lambda i: (i, 0)),
    pl.BlockSpec((1, _D), lambda i: (0, 0)),
    pl.BlockSpec((2 * _K + _D, _D), lambda i: (0, 0)),
    pl.BlockSpec((1, _D), lambda i: (0, 0)),
]


def _tc_comb0(ut, s, cs, aggp, dinv, bc, Wd, bd):
    return pl.pallas_call(
        _comb0_body,
        grid=(_NBLK,),
        in_specs=_comb_in_specs,
        out_specs=[
            pl.BlockSpec((_R, _D), lambda i: (i, 0)),
            pl.BlockSpec((2, _D), lambda i: (0, 0)),
        ],
        out_shape=[
            jax.ShapeDtypeStruct((_N, _D), jnp.float32),
            jax.ShapeDtypeStruct((2, _D), jnp.float32),
        ],
    )(ut, s, cs, aggp, dinv, bc, Wd, bd)


def _tc_comb1(ut, s, cs, aggp, dinv, bc, Wd, bd):
    return pl.pallas_call(
        _comb1_body,
        grid=(_NBLK,),
        in_specs=_comb_in_specs,
        out_specs=pl.BlockSpec((_R, _D), lambda i: (i, 0)),
        out_shape=jax.ShapeDtypeStruct((_N, _D), jnp.float32),
    )(ut, s, cs, aggp, dinv, bc, Wd, bd)


def _pred_body(g0_ref, g1_ref, W0_ref, b0_ref, W1_ref, b1_ref, out_ref):
    m = g0_ref[...] * g1_ref[...]
    h = jax.nn.relu(jnp.dot(m, W0_ref[...], preferred_element_type=jnp.float32)
                    + b0_ref[...])
    o = jnp.dot(h, W1_ref[...], preferred_element_type=jnp.float32) + b1_ref[...]
    out_ref[...] = jax.nn.sigmoid(o)


_BP = 4096


def _tc_pred(g0, g1, W0, b0, W1, b1):
    return pl.pallas_call(
        _pred_body,
        grid=(_NE // _BP,),
        in_specs=[
            pl.BlockSpec((_BP, _D), lambda i: (i, 0)),
            pl.BlockSpec((_BP, _D), lambda i: (i, 0)),
            pl.BlockSpec((_D, _D), lambda i: (0, 0)),
            pl.BlockSpec((1, _D), lambda i: (0, 0)),
            pl.BlockSpec((_D, 1), lambda i: (0, 0)),
            pl.BlockSpec((1, 1), lambda i: (0, 0)),
        ],
        out_specs=pl.BlockSpec((_BP, 1), lambda i: (i, 0)),
        out_shape=jax.ShapeDtypeStruct((_NE, 1), jnp.float32),
    )(g0, g1, W0, b0, W1, b1)


def kernel(adj_t, edges, emb, conv_W0, conv_b0, attn_W0, attn_b0, dr_W0,
           dr_b0, bn_g0, bn_b0, conv_W1, conv_b1, attn_W1, attn_b1, dr_W1,
           dr_b1, pred_W0, pred_b0, pred_W1, pred_b1):
    pad = _EPAD - _EPW
    srcp = jnp.concatenate(
        [adj_t[0].reshape(_NW, _EPW),
         jnp.zeros((_NW, pad), jnp.int32)], axis=1)
    dstp = jnp.concatenate(
        [adj_t[1].reshape(_NW, _EPW),
         jnp.full((_NW, pad), _N, jnp.int32)], axis=1)
    srcI = srcp.reshape(_NW, _NCHUNK, _ECH)
    dstI = dstp.reshape(_NW, _NCHUNK, _ECH)
    dst16 = dstp.reshape(_NW, _EPAD // 16, 16)
    e0I = edges[0].reshape(_NW, _NEPW // _CHUNK, _CHUNK)
    e1I = edges[1].reshape(_NW, _NEPW // _CHUNK, _CHUNK)
    zerosA = jnp.zeros((_NAGG // _NS, _D), jnp.float32)

    bc0 = conv_b0.reshape(1, _D)
    bc1 = conv_b1.reshape(1, _D)
    ba0 = attn_b0.reshape(1, 4 * _K)
    ba1 = attn_b1.reshape(1, 4 * _K)
    bd0 = dr_b0.reshape(1, _D)
    bd1 = dr_b1.reshape(1, _D)
    bng = bn_g0.reshape(1, _D)
    bnb = bn_b0.reshape(1, _D)
    bp0 = pred_b0.reshape(1, _D)
    bp1 = pred_b1.reshape(1, 1)

    degp = _sc_deg()(dst16)

    hn0, dinv = _tc_conv0(emb, degp.T, conv_W0)
    aggp0 = _sc_agg()(hn0, srcI, dstI, zerosA)
    ut0, s0, cs0 = _tc_attn0(emb, attn_W0, ba0)
    y0, bnstats = _tc_comb0(ut0, s0, cs0, aggp0, dinv, bc0, dr_W0, bd0)

    hn1 = _tc_conv1(y0, bnstats, bng, bnb, dinv, conv_W1)
    aggp1 = _sc_agg()(hn1, srcI, dstI, zerosA)
    ut1, s1, cs1 = _tc_attn1(y0, bnstats, bng, bnb, attn_W1, ba1)
    xf = _tc_comb1(ut1, s1, cs1, aggp1, dinv, bc1, dr_W1, bd1)

    g0, g1 = _sc_pairs()(xf, e0I, e1I)
    return _tc_pred(g0, g1, pred_W0, bp0, pred_W1, bp1)
